# Initial kernel scaffold; baseline (speedup 1.0000x reference)
import jax
import jax.numpy as jnp
from jax.experimental import pallas as pl


def kernel(x, edge_index, W1, b1, Wu, bu):
    raise NotImplementedError("write your pallas kernel here")


# speedup vs baseline: 5.7884x; 5.7884x over previous
import functools

import jax
import jax.numpy as jnp
from jax import lax
from jax.experimental import pallas as pl
from jax.experimental.pallas import tpu as pltpu
from jax.experimental.pallas import tpu_sc as plsc

N = 10000
E = 10000
NI = 320000
D = 128
DP = 144
NC = 2
NS = 16
CHUNK = 128
PER_CORE = NI // NC
NCHUNK = PER_CORE // CHUNK
ZROWS = 80
NZBLK = N // ZROWS
BLK = 1000


def _sc_segment_sum(table, gidx, sidx):
    mesh = plsc.VectorSubcoreMesh(core_axis_name="c", subcore_axis_name="s")

    @functools.partial(
        pl.kernel,
        out_type=jax.ShapeDtypeStruct((NC, N, DP), jnp.float32),
        mesh=mesh,
        scratch_types=[
            pltpu.VMEM((1, CHUNK), jnp.int32),
            pltpu.VMEM((1, CHUNK), jnp.int32),
            pltpu.VMEM((CHUNK, DP), jnp.float32),
            pltpu.VMEM((CHUNK, DP), jnp.float32),
            pltpu.VMEM_SHARED((N, DP), jnp.float32),
            pltpu.SemaphoreType.DMA,
        ],
        compiler_params=pltpu.CompilerParams(use_tc_tiling_on_sc=False),
    )
    def seg_kernel(table_hbm, gidx_hbm, sidx_hbm, out_hbm,
                   gi_v, si_v, b0_v, b1_v, acc_sh, sem):
        c = lax.axis_index("c")
        s = lax.axis_index("s")

        @pl.loop(0, ZROWS)
        def _(r):
            for j in range(DP // 16):
                b0_v[r, pl.ds(j * 16, 16)] = jnp.zeros((16,), jnp.float32)

        @pl.loop(s, NZBLK, step=NS)
        def _(t):
            pltpu.sync_copy(b0_v.at[pl.ds(0, ZROWS)],
                            acc_sh.at[pl.ds(t * ZROWS, ZROWS)])

        plsc.subcore_barrier()

        @pl.loop(s, NCHUNK, step=NS)
        def _(q):
            base = c * PER_CORE + q * CHUNK
            pltpu.sync_copy(gidx_hbm.at[pl.ds(base, CHUNK)], gi_v.at[0])
            pltpu.sync_copy(sidx_hbm.at[pl.ds(base, CHUNK)], si_v.at[0])
            pltpu.async_copy(table_hbm.at[gi_v.at[0]], b0_v, sem).wait()
            pltpu.sync_copy(b0_v, acc_sh.at[si_v.at[0]], add=True)

        plsc.subcore_barrier()

        @pl.loop(s, NZBLK, step=NS)
        def _(t):
            row0 = t * ZROWS
            pltpu.sync_copy(acc_sh.at[pl.ds(row0, ZROWS)],
                            b1_v.at[pl.ds(0, ZROWS)])
            pltpu.sync_copy(b1_v.at[pl.ds(0, ZROWS)],
                            out_hbm.at[c, pl.ds(row0, ZROWS)])

    return seg_kernel(table, gidx, sidx)


def _ones_pad(nrows):
    col = lax.broadcasted_iota(jnp.int32, (nrows, DP - D), 1)
    return jnp.where(col == 0, 1.0, 0.0).astype(jnp.float32)


def _tc_front(x, W1, b1, Wu, bu):
    def body(x_ref, w1_ref, b1_ref, wu_ref, bu_ref, maug_ref, u_ref):
        xb = x_ref[...]
        m = jnp.maximum(
            jnp.dot(xb, w1_ref[...], preferred_element_type=jnp.float32)
            + b1_ref[...], 0.0)
        maug_ref[:, :D] = m
        maug_ref[:, D:] = _ones_pad(BLK)
        u_ref[...] = (
            jnp.dot(xb, wu_ref[...], preferred_element_type=jnp.float32)
            + bu_ref[...])

    return pl.pallas_call(
        body,
        grid=(N // BLK,),
        in_specs=[
            pl.BlockSpec((BLK, D), lambda i: (i, 0)),
            pl.BlockSpec((D, D), lambda i: (0, 0)),
            pl.BlockSpec((1, D), lambda i: (0, 0)),
            pl.BlockSpec((D, D), lambda i: (0, 0)),
            pl.BlockSpec((1, D), lambda i: (0, 0)),
        ],
        out_specs=[
            pl.BlockSpec((BLK, DP), lambda i: (i, 0)),
            pl.BlockSpec((BLK, D), lambda i: (i, 0)),
        ],
        out_shape=[
            jax.ShapeDtypeStruct((N, DP), jnp.float32),
            jax.ShapeDtypeStruct((N, D), jnp.float32),
        ],
    )(x, W1, b1.reshape(1, D), Wu, bu.reshape(1, D))


def _tc_mid(parts):
    def body(p_ref, o_ref):
        ssum = p_ref[0] + p_ref[1]
        cnt = ssum[:, D:D + 1]
        o_ref[:, :D] = ssum[:, :D] / jnp.maximum(cnt, 1.0)
        o_ref[:, D:] = _ones_pad(BLK)

    return pl.pallas_call(
        body,
        grid=(E // BLK,),
        in_specs=[pl.BlockSpec((NC, BLK, DP), lambda i: (0, i, 0))],
        out_specs=pl.BlockSpec((BLK, DP), lambda i: (i, 0)),
        out_shape=jax.ShapeDtypeStruct((E, DP), jnp.float32),
    )(parts)


def _tc_back(u, parts):
    def body(u_ref, p_ref, o_ref):
        ssum = p_ref[0] + p_ref[1]
        cnt = ssum[:, D:D + 1]
        mi = ssum[:, :D] / jnp.maximum(cnt, 1.0)
        h = jnp.maximum(u_ref[...] + mi, 0.0)
        mx = jnp.max(h, axis=1, keepdims=True)
        lse = jnp.log(jnp.sum(jnp.exp(h - mx), axis=1, keepdims=True))
        o_ref[...] = h - mx - lse

    return pl.pallas_call(
        body,
        grid=(N // BLK,),
        in_specs=[
            pl.BlockSpec((BLK, D), lambda i: (i, 0)),
            pl.BlockSpec((NC, BLK, DP), lambda i: (0, i, 0)),
        ],
        out_specs=pl.BlockSpec((BLK, D), lambda i: (i, 0)),
        out_shape=jax.ShapeDtypeStruct((N, D), jnp.float32),
    )(u, parts)


def kernel(x, edge_index, W1, b1, Wu, bu):
    v_idx = edge_index[0]
    e_idx = edge_index[1]
    maug, u = _tc_front(x, W1, b1, Wu, bu)
    e_parts = _sc_segment_sum(maug, v_idx, e_idx)
    e_feat = _tc_mid(e_parts)
    v_parts = _sc_segment_sum(e_feat, e_idx, v_idx)
    return _tc_back(u, v_parts)


# speedup vs baseline: 9.5348x; 1.6472x over previous
import functools

import jax
import jax.numpy as jnp
from jax import lax
from jax.experimental import pallas as pl
from jax.experimental.pallas import tpu as pltpu
from jax.experimental.pallas import tpu_sc as plsc

N = 10000
E = 10000
NI = 320000
D = 128
DP = 144
NC = 2
NS = 16
CHUNK = 128
PER_CORE = NI // NC
NCHUNK = PER_CORE // CHUNK
ZROWS = 80
NZBLK = N // ZROWS
BLK = 1000


def _sc_segment_sum(table, gidx, sidx):
    mesh = plsc.VectorSubcoreMesh(core_axis_name="c", subcore_axis_name="s")

    @functools.partial(
        pl.kernel,
        out_type=jax.ShapeDtypeStruct((NC, N, DP), jnp.float32),
        mesh=mesh,
        scratch_types=[
            pltpu.VMEM((2, CHUNK), jnp.int32),
            pltpu.VMEM((2, CHUNK), jnp.int32),
            pltpu.VMEM((CHUNK, DP), jnp.float32),
            pltpu.VMEM((CHUNK, DP), jnp.float32),
            pltpu.VMEM_SHARED((N, DP), jnp.float32),
            pltpu.SemaphoreType.DMA,
            pltpu.SemaphoreType.DMA,
            pltpu.SemaphoreType.DMA,
            pltpu.SemaphoreType.DMA,
        ],
        compiler_params=pltpu.CompilerParams(use_tc_tiling_on_sc=False),
    )
    def seg_kernel(table_hbm, gidx_hbm, sidx_hbm, out_hbm,
                   gi_v, si_v, b0_v, b1_v, acc_sh,
                   gsem0, gsem1, isem0, isem1):
        c = lax.axis_index("c")
        s = lax.axis_index("s")
        bufs = (b0_v, b1_v)
        gsems = (gsem0, gsem1)
        isems = (isem0, isem1)

        @pl.loop(0, ZROWS)
        def _(r):
            for j in range(DP // 16):
                b0_v[r, pl.ds(j * 16, 16)] = jnp.zeros((16,), jnp.float32)

        @pl.loop(s, NZBLK, step=NS)
        def _(t):
            pltpu.sync_copy(b0_v.at[pl.ds(0, ZROWS)],
                            acc_sh.at[pl.ds(t * ZROWS, ZROWS)])

        plsc.subcore_barrier()

        def chunk_base(i):
            return c * PER_CORE + (s + i * NS) * CHUNK

        def start_idx_fetch(i, slot):
            base = chunk_base(i)
            pltpu.async_copy(gidx_hbm.at[pl.ds(base, CHUNK)],
                             gi_v.at[slot], isems[slot])
            pltpu.async_copy(sidx_hbm.at[pl.ds(base, CHUNK)],
                             si_v.at[slot], isems[slot])

        def wait_idx(slot):
            pltpu.make_async_copy(gidx_hbm.at[pl.ds(0, CHUNK)],
                                  gi_v.at[slot], isems[slot]).wait()
            pltpu.make_async_copy(gidx_hbm.at[pl.ds(0, CHUNK)],
                                  si_v.at[slot], isems[slot]).wait()

        def start_gather(slot):
            pltpu.async_copy(table_hbm.at[gi_v.at[slot]],
                             bufs[slot], gsems[slot])

        def wait_gather(slot):
            pltpu.make_async_copy(table_hbm.at[pl.ds(0, CHUNK)],
                                  bufs[slot], gsems[slot]).wait()

        def scatter_add(slot):
            pltpu.sync_copy(bufs[slot], acc_sh.at[si_v.at[slot]], add=True)

        base0 = chunk_base(0)
        pltpu.sync_copy(gidx_hbm.at[pl.ds(base0, CHUNK)], gi_v.at[0])
        pltpu.sync_copy(sidx_hbm.at[pl.ds(base0, CHUNK)], si_v.at[0])
        start_idx_fetch(1, 1)
        start_gather(0)

        @pl.loop(0, 38)
        def _(p):
            i0 = 2 * p
            wait_idx(1)
            start_gather(1)
            wait_gather(0)
            scatter_add(0)
            start_idx_fetch(i0 + 2, 0)
            wait_idx(0)
            start_gather(0)
            wait_gather(1)
            scatter_add(1)
            start_idx_fetch(i0 + 3, 1)

        wait_idx(1)
        start_gather(1)
        wait_gather(0)
        scatter_add(0)
        wait_gather(1)
        scatter_add(1)

        @pl.when(s < 2)
        def _():
            base = c * PER_CORE + (NCHUNK - 2 + s) * CHUNK
            pltpu.sync_copy(gidx_hbm.at[pl.ds(base, CHUNK)], gi_v.at[0])
            pltpu.sync_copy(sidx_hbm.at[pl.ds(base, CHUNK)], si_v.at[0])
            pltpu.async_copy(table_hbm.at[gi_v.at[0]], b0_v, gsem0).wait()
            scatter_add(0)

        plsc.subcore_barrier()

        @pl.loop(s, NZBLK, step=NS)
        def _(t):
            row0 = t * ZROWS
            pltpu.sync_copy(acc_sh.at[pl.ds(row0, ZROWS)],
                            b1_v.at[pl.ds(0, ZROWS)])
            pltpu.sync_copy(b1_v.at[pl.ds(0, ZROWS)],
                            out_hbm.at[c, pl.ds(row0, ZROWS)])

    return seg_kernel(table, gidx, sidx)


def _ones_pad(nrows):
    col = lax.broadcasted_iota(jnp.int32, (nrows, DP - D), 1)
    return jnp.where(col == 0, 1.0, 0.0).astype(jnp.float32)


def _tc_front(x, W1, b1, Wu, bu):
    def body(x_ref, w1_ref, b1_ref, wu_ref, bu_ref, maug_ref, u_ref):
        xb = x_ref[...]
        m = jnp.maximum(
            jnp.dot(xb, w1_ref[...], preferred_element_type=jnp.float32)
            + b1_ref[...], 0.0)
        maug_ref[:, :D] = m
        maug_ref[:, D:] = _ones_pad(BLK)
        u_ref[...] = (
            jnp.dot(xb, wu_ref[...], preferred_element_type=jnp.float32)
            + bu_ref[...])

    return pl.pallas_call(
        body,
        grid=(N // BLK,),
        in_specs=[
            pl.BlockSpec((BLK, D), lambda i: (i, 0)),
            pl.BlockSpec((D, D), lambda i: (0, 0)),
            pl.BlockSpec((1, D), lambda i: (0, 0)),
            pl.BlockSpec((D, D), lambda i: (0, 0)),
            pl.BlockSpec((1, D), lambda i: (0, 0)),
        ],
        out_specs=[
            pl.BlockSpec((BLK, DP), lambda i: (i, 0)),
            pl.BlockSpec((BLK, D), lambda i: (i, 0)),
        ],
        out_shape=[
            jax.ShapeDtypeStruct((N, DP), jnp.float32),
            jax.ShapeDtypeStruct((N, D), jnp.float32),
        ],
    )(x, W1, b1.reshape(1, D), Wu, bu.reshape(1, D))


def _tc_mid(parts):
    def body(p_ref, o_ref):
        ssum = p_ref[0] + p_ref[1]
        cnt = ssum[:, D:D + 1]
        o_ref[:, :D] = ssum[:, :D] / jnp.maximum(cnt, 1.0)
        o_ref[:, D:] = _ones_pad(BLK)

    return pl.pallas_call(
        body,
        grid=(E // BLK,),
        in_specs=[pl.BlockSpec((NC, BLK, DP), lambda i: (0, i, 0))],
        out_specs=pl.BlockSpec((BLK, DP), lambda i: (i, 0)),
        out_shape=jax.ShapeDtypeStruct((E, DP), jnp.float32),
    )(parts)


def _tc_back(u, parts):
    def body(u_ref, p_ref, o_ref):
        ssum = p_ref[0] + p_ref[1]
        cnt = ssum[:, D:D + 1]
        mi = ssum[:, :D] / jnp.maximum(cnt, 1.0)
        h = jnp.maximum(u_ref[...] + mi, 0.0)
        mx = jnp.max(h, axis=1, keepdims=True)
        lse = jnp.log(jnp.sum(jnp.exp(h - mx), axis=1, keepdims=True))
        o_ref[...] = h - mx - lse

    return pl.pallas_call(
        body,
        grid=(N // BLK,),
        in_specs=[
            pl.BlockSpec((BLK, D), lambda i: (i, 0)),
            pl.BlockSpec((NC, BLK, DP), lambda i: (0, i, 0)),
        ],
        out_specs=pl.BlockSpec((BLK, D), lambda i: (i, 0)),
        out_shape=jax.ShapeDtypeStruct((N, D), jnp.float32),
    )(u, parts)


def kernel(x, edge_index, W1, b1, Wu, bu):
    v_idx = edge_index[0]
    e_idx = edge_index[1]
    maug, u = _tc_front(x, W1, b1, Wu, bu)
    e_parts = _sc_segment_sum(maug, v_idx, e_idx)
    e_feat = _tc_mid(e_parts)
    v_parts = _sc_segment_sum(e_feat, e_idx, v_idx)
    return _tc_back(u, v_parts)
